# Initial kernel scaffold; baseline (speedup 1.0000x reference)
import jax
import jax.numpy as jnp
from jax.experimental import pallas as pl


def kernel(x, edge_index, edge_features, W1, b1, W2, b2, W3, b3, W4, b4):
    raise NotImplementedError("write your pallas kernel here")


# speedup vs baseline: 10.1712x; 10.1712x over previous
import functools

import jax
import jax.numpy as jnp
from jax import lax
from jax.experimental import pallas as pl
from jax.experimental.pallas import tpu as pltpu
from jax.experimental.pallas import tpu_sc as plsc

_NC = 2
_NS = 16
_NW = _NC * _NS

_SQRT_HALF = 0.7071067811865476


def _gelu(t):
    return 0.5 * t * (1.0 + lax.erf(t * _SQRT_HALF))


def _proj_nodes_body(x_ref, w_ref, psrc_ref, pdst_ref):
    p = lax.dot_general(x_ref[...], w_ref[...], (((1,), (1,)), ((), ())),
                        preferred_element_type=jnp.float32)
    de = psrc_ref.shape[-1]
    psrc_ref[...] = p[:, :de]
    pdst_ref[...] = p[:, de:]


def _edge_proj_body(ef_ref, k_ref, b_ref, out_ref):
    out_ref[...] = (
        lax.dot_general(ef_ref[...], k_ref[...], (((1,), (0,)), ((), ())),
                        preferred_element_type=jnp.float32)
        + b_ref[...]
    )


def _edge_mlp2_body(h_ref, k_ref, b_ref, out_ref):
    g = _gelu(h_ref[...])
    out_ref[...] = (
        lax.dot_general(g, k_ref[...], (((1,), (0,)), ((), ())),
                        preferred_element_type=jnp.float32)
        + b_ref[...]
    )


def _node_mlp_body(x_ref, p_ref, w3a_ref, w3b_ref, b3_ref, w4_ref, b4_ref,
                   out_ref):
    x = x_ref[...]
    agg = p_ref[0] + p_ref[1]
    t = (
        lax.dot_general(x, w3a_ref[...], (((1,), (1,)), ((), ())),
                        preferred_element_type=jnp.float32)
        + lax.dot_general(agg, w3b_ref[...], (((1,), (1,)), ((), ())),
                          preferred_element_type=jnp.float32)
        + b3_ref[...]
    )
    h2 = _gelu(t)
    out_ref[...] = (
        x
        + lax.dot_general(h2, w4_ref[...], (((1,), (1,)), ((), ())),
                          preferred_element_type=jnp.float32)
        + b4_ref[...]
    )


_SUP = 1000
_CH = 125
_NCH = _SUP // _CH


def _make_gather_kernel(n_nodes, n_edges, de):
    ew = n_edges // _NW
    nsup = ew // _SUP
    mesh = plsc.VectorSubcoreMesh(core_axis_name="c", subcore_axis_name="s",
                                  num_cores=_NC, num_subcores=_NS)

    @functools.partial(
        pl.kernel,
        mesh=mesh,
        out_type=jax.ShapeDtypeStruct((_NW, ew, de), jnp.float32),
        scratch_types=[
            pltpu.VMEM((_NCH, _CH), jnp.int32),
            pltpu.VMEM((_NCH, _CH), jnp.int32),
            pltpu.VMEM((_SUP, de), jnp.float32),
            pltpu.VMEM((_SUP, de), jnp.float32),
            pltpu.VMEM((_SUP, de), jnp.float32),
            pltpu.SemaphoreType.DMA,
        ],
        compiler_params=pltpu.CompilerParams(use_tc_tiling_on_sc=False),
    )
    def gather_k(psrc_hbm, pdst_hbm, eproj_hbm, sidx_hbm, didx_hbm, out_hbm,
                 sidx_v, didx_v, h_v, rs_v, rd_v, sem):
        wid = lax.axis_index("s") * _NC + lax.axis_index("c")

        def super_body(s, carry):
            irow = pl.multiple_of(s * _NCH, 8)
            ebase = pl.multiple_of(s * _SUP, 8)
            pltpu.sync_copy(sidx_hbm.at[wid, pl.ds(irow, _NCH), :], sidx_v)
            pltpu.sync_copy(didx_hbm.at[wid, pl.ds(irow, _NCH), :], didx_v)
            pltpu.sync_copy(eproj_hbm.at[wid, pl.ds(ebase, _SUP), :], h_v)
            descs = []
            for c in range(_NCH):
                descs.append(pltpu.async_copy(
                    psrc_hbm.at[sidx_v.at[c]],
                    rs_v.at[pl.ds(c * _CH, _CH), :], sem))
                descs.append(pltpu.async_copy(
                    pdst_hbm.at[didx_v.at[c]],
                    rd_v.at[pl.ds(c * _CH, _CH), :], sem))
            for d in descs:
                d.wait()

            def row_body(j, c):
                h_v[j, :] = h_v[j, :] + rs_v[j, :] + rd_v[j, :]
                return c

            lax.fori_loop(0, _SUP, row_body, 0, unroll=4)
            pltpu.sync_copy(h_v, out_hbm.at[wid, pl.ds(ebase, _SUP), :])
            return carry

        lax.fori_loop(0, nsup, super_body, 0)

    return gather_k


def _make_scatter_kernel(n_pad, n_edges, de):
    ew = n_edges // _NW
    nsup = ew // _SUP
    nrows = n_pad // _NS
    mesh = plsc.VectorSubcoreMesh(core_axis_name="c", subcore_axis_name="s",
                                  num_cores=_NC, num_subcores=_NS)

    @functools.partial(
        pl.kernel,
        mesh=mesh,
        out_type=jax.ShapeDtypeStruct((_NC, n_pad, de), jnp.float32),
        scratch_types=[
            pltpu.VMEM((_NCH, _CH), jnp.int32),
            pltpu.VMEM((_SUP, de), jnp.float32),
            pltpu.VMEM_SHARED((n_pad, de), jnp.float32),
        ],
        compiler_params=pltpu.CompilerParams(use_tc_tiling_on_sc=False),
    )
    def scatter_k(msg_hbm, didx_hbm, out_hbm, didx_v, msg_v, acc_sh):
        cid = lax.axis_index("c")
        sid = lax.axis_index("s")
        wid = sid * _NC + cid
        rbase = pl.multiple_of(sid * nrows, 8)

        def zrow(j, c):
            msg_v[j, :] = jnp.zeros((de,), jnp.float32)
            return c

        lax.fori_loop(0, nrows, zrow, 0)
        pltpu.sync_copy(msg_v.at[pl.ds(0, nrows), :],
                        acc_sh.at[pl.ds(rbase, nrows), :])
        plsc.subcore_barrier()

        def super_body(s, carry):
            irow = pl.multiple_of(s * _NCH, 8)
            ebase = pl.multiple_of(s * _SUP, 8)
            pltpu.sync_copy(didx_hbm.at[wid, pl.ds(irow, _NCH), :], didx_v)
            pltpu.sync_copy(msg_hbm.at[wid, pl.ds(ebase, _SUP), :], msg_v)
            for c in range(_NCH):
                pltpu.sync_copy(msg_v.at[pl.ds(c * _CH, _CH), :],
                                acc_sh.at[didx_v.at[c]], add=True)
            return carry

        lax.fori_loop(0, nsup, super_body, 0)
        plsc.subcore_barrier()
        pltpu.sync_copy(acc_sh.at[pl.ds(rbase, nrows), :],
                        out_hbm.at[cid, pl.ds(rbase, nrows), :])

    return scatter_k


def kernel(x, edge_index, edge_features, W1, b1, W2, b2, W3, b3, W4, b4):
    bsz, n, d = x.shape
    e = edge_index.shape[1]
    de = edge_features.shape[1]
    assert bsz == 1 and d == 128 and de == 16
    assert e % (_NW * _SUP) == 0 and n % _NS == 0 and (n * de) % 128 == 0

    w1ab = jnp.concatenate([W1[:, :d], W1[:, d:2 * d]], axis=0)
    w1c_t = W1[:, 2 * d:].T
    eye8 = jnp.eye(8, dtype=jnp.float32)
    k1 = jnp.kron(eye8, w1c_t)
    b1t = jnp.tile(b1, 8)[None, :]
    k2 = jnp.kron(eye8, W2.T)
    b2t = jnp.tile(b2, 8)[None, :]
    w3a = W3[:, :d]
    w3b = W3[:, d:]
    b3t = b3[None, :]
    b4t = b4[None, :]

    x2d = x[0]
    ew = e // _NW
    sidx = edge_index[0].reshape(_NW, ew // _CH, _CH)
    didx = edge_index[1].reshape(_NW, ew // _CH, _CH)
    ef_p = edge_features.reshape(e // 8, 8 * de)
    n_pad = ((n + _NS * 8 - 1) // (_NS * 8)) * _NS * 8

    f32 = jnp.float32
    row_blk = 1000
    ep_blk = 2000
    n_grid = n // row_blk
    ep_grid = (e // 8) // ep_blk

    psrc, pdst = pl.pallas_call(
        _proj_nodes_body,
        grid=(n_grid,),
        in_specs=[
            pl.BlockSpec((row_blk, d), lambda i: (i, 0)),
            pl.BlockSpec((2 * de, d), lambda i: (0, 0)),
        ],
        out_specs=[
            pl.BlockSpec((row_blk, de), lambda i: (i, 0)),
            pl.BlockSpec((row_blk, de), lambda i: (i, 0)),
        ],
        out_shape=[jax.ShapeDtypeStruct((n, de), f32)] * 2,
    )(x2d, w1ab)

    eproj_p = pl.pallas_call(
        _edge_proj_body,
        grid=(ep_grid,),
        in_specs=[
            pl.BlockSpec((ep_blk, 8 * de), lambda i: (i, 0)),
            pl.BlockSpec((8 * de, 8 * de), lambda i: (0, 0)),
            pl.BlockSpec((1, 8 * de), lambda i: (0, 0)),
        ],
        out_specs=pl.BlockSpec((ep_blk, 8 * de), lambda i: (i, 0)),
        out_shape=jax.ShapeDtypeStruct((e // 8, 8 * de), f32),
    )(ef_p, k1, b1t)
    eproj = eproj_p.reshape(_NW, ew, de)

    h = _make_gather_kernel(n, e, de)(psrc, pdst, eproj, sidx, didx)

    msg_p = pl.pallas_call(
        _edge_mlp2_body,
        grid=(ep_grid,),
        in_specs=[
            pl.BlockSpec((ep_blk, 8 * de), lambda i: (i, 0)),
            pl.BlockSpec((8 * de, 8 * de), lambda i: (0, 0)),
            pl.BlockSpec((1, 8 * de), lambda i: (0, 0)),
        ],
        out_specs=pl.BlockSpec((ep_blk, 8 * de), lambda i: (i, 0)),
        out_shape=jax.ShapeDtypeStruct((e // 8, 8 * de), f32),
    )(h.reshape(e // 8, 8 * de), k2, b2t)
    msg = msg_p.reshape(_NW, ew, de)

    partials = _make_scatter_kernel(n_pad, e, de)(msg, didx)

    out2d = pl.pallas_call(
        _node_mlp_body,
        grid=(n_grid,),
        in_specs=[
            pl.BlockSpec((row_blk, d), lambda i: (i, 0)),
            pl.BlockSpec((_NC, row_blk, de), lambda i: (0, i, 0)),
            pl.BlockSpec((d, d), lambda i: (0, 0)),
            pl.BlockSpec((d, de), lambda i: (0, 0)),
            pl.BlockSpec((1, d), lambda i: (0, 0)),
            pl.BlockSpec((d, d), lambda i: (0, 0)),
            pl.BlockSpec((1, d), lambda i: (0, 0)),
        ],
        out_specs=pl.BlockSpec((row_blk, d), lambda i: (i, 0)),
        out_shape=jax.ShapeDtypeStruct((n, d), f32),
    )(x2d, partials, w3a, w3b, b3t, W4, b4t)

    return out2d[None]


# speedup vs baseline: 11.6654x; 1.1469x over previous
import functools

import jax
import jax.numpy as jnp
from jax import lax
from jax.experimental import pallas as pl
from jax.experimental.pallas import tpu as pltpu
from jax.experimental.pallas import tpu_sc as plsc

_NC = 2
_NS = 16
_NW = _NC * _NS

_SQRT_HALF = 0.7071067811865476


def _gelu(t):
    return 0.5 * t * (1.0 + lax.erf(t * _SQRT_HALF))


def _proj_nodes_body(x_ref, w_ref, psrc_ref, pdst_ref):
    p = lax.dot_general(x_ref[...], w_ref[...], (((1,), (1,)), ((), ())),
                        preferred_element_type=jnp.float32)
    de = psrc_ref.shape[-1]
    psrc_ref[...] = p[:, :de]
    pdst_ref[...] = p[:, de:]


def _edge_proj_body(ef_ref, k_ref, b_ref, out_ref):
    out_ref[...] = (
        lax.dot_general(ef_ref[...], k_ref[...], (((1,), (0,)), ((), ())),
                        preferred_element_type=jnp.float32)
        + b_ref[...]
    )


def _edge_mlp2_body(h_ref, k_ref, b_ref, out_ref):
    g = _gelu(h_ref[...])
    out_ref[...] = (
        lax.dot_general(g, k_ref[...], (((1,), (0,)), ((), ())),
                        preferred_element_type=jnp.float32)
        + b_ref[...]
    )


def _node_mlp_body(x_ref, p_ref, w3a_ref, w3b_ref, b3_ref, w4_ref, b4_ref,
                   out_ref):
    x = x_ref[...]
    agg = p_ref[0] + p_ref[1]
    t = (
        lax.dot_general(x, w3a_ref[...], (((1,), (1,)), ((), ())),
                        preferred_element_type=jnp.float32)
        + lax.dot_general(agg, w3b_ref[...], (((1,), (1,)), ((), ())),
                          preferred_element_type=jnp.float32)
        + b3_ref[...]
    )
    h2 = _gelu(t)
    out_ref[...] = (
        x
        + lax.dot_general(h2, w4_ref[...], (((1,), (1,)), ((), ())),
                          preferred_element_type=jnp.float32)
        + b4_ref[...]
    )


_SUP = 1000
_CH = 125
_NCH = _SUP // _CH


def _make_gather_kernel(n_nodes, n_edges, de):
    ew = n_edges // _NW
    nsup = ew // _SUP
    mesh = plsc.VectorSubcoreMesh(core_axis_name="c", subcore_axis_name="s",
                                  num_cores=_NC, num_subcores=_NS)

    @functools.partial(
        pl.kernel,
        mesh=mesh,
        out_type=jax.ShapeDtypeStruct((_NW, ew, de), jnp.float32),
        scratch_types=[
            pltpu.VMEM((_NCH, _CH), jnp.int32),
            pltpu.VMEM((_NCH, _CH), jnp.int32),
            pltpu.VMEM((_SUP, de), jnp.float32),
            pltpu.SemaphoreType.DMA,
        ],
        compiler_params=pltpu.CompilerParams(use_tc_tiling_on_sc=False),
    )
    def gather_k(psrc_hbm, pdst_hbm, eproj_hbm, sidx_hbm, didx_hbm, out_hbm,
                 sidx_v, didx_v, h_v, sem):
        wid = lax.axis_index("s") * _NC + lax.axis_index("c")

        def super_body(s, carry):
            irow = pl.multiple_of(s * _NCH, 8)
            ebase = pl.multiple_of(s * _SUP, 8)
            pltpu.sync_copy(sidx_hbm.at[wid, pl.ds(irow, _NCH), :], sidx_v)
            pltpu.sync_copy(didx_hbm.at[wid, pl.ds(irow, _NCH), :], didx_v)
            pltpu.sync_copy(eproj_hbm.at[wid, pl.ds(ebase, _SUP), :], h_v)
            descs = []
            for c in range(_NCH):
                descs.append(pltpu.async_copy(
                    psrc_hbm.at[sidx_v.at[c]],
                    h_v.at[pl.ds(c * _CH, _CH), :], sem, add=True))
            for d in descs:
                d.wait()
            descs = []
            for c in range(_NCH):
                descs.append(pltpu.async_copy(
                    pdst_hbm.at[didx_v.at[c]],
                    h_v.at[pl.ds(c * _CH, _CH), :], sem, add=True))
            for d in descs:
                d.wait()
            pltpu.sync_copy(h_v, out_hbm.at[wid, pl.ds(ebase, _SUP), :])
            return carry

        lax.fori_loop(0, nsup, super_body, 0)

    return gather_k


def _make_scatter_kernel(n_pad, n_edges, de):
    ew = n_edges // _NW
    nsup = ew // _SUP
    nrows = n_pad // _NS
    mesh = plsc.VectorSubcoreMesh(core_axis_name="c", subcore_axis_name="s",
                                  num_cores=_NC, num_subcores=_NS)

    @functools.partial(
        pl.kernel,
        mesh=mesh,
        out_type=jax.ShapeDtypeStruct((_NC, n_pad, de), jnp.float32),
        scratch_types=[
            pltpu.VMEM((_NCH, _CH), jnp.int32),
            pltpu.VMEM((_SUP, de), jnp.float32),
            pltpu.VMEM_SHARED((n_pad, de), jnp.float32),
        ],
        compiler_params=pltpu.CompilerParams(use_tc_tiling_on_sc=False),
    )
    def scatter_k(msg_hbm, didx_hbm, out_hbm, didx_v, msg_v, acc_sh):
        cid = lax.axis_index("c")
        sid = lax.axis_index("s")
        wid = sid * _NC + cid
        rbase = pl.multiple_of(sid * nrows, 8)

        def zrow(j, c):
            msg_v[j, :] = jnp.zeros((de,), jnp.float32)
            return c

        lax.fori_loop(0, nrows, zrow, 0)
        pltpu.sync_copy(msg_v.at[pl.ds(0, nrows), :],
                        acc_sh.at[pl.ds(rbase, nrows), :])
        plsc.subcore_barrier()

        def super_body(s, carry):
            irow = pl.multiple_of(s * _NCH, 8)
            ebase = pl.multiple_of(s * _SUP, 8)
            pltpu.sync_copy(didx_hbm.at[wid, pl.ds(irow, _NCH), :], didx_v)
            pltpu.sync_copy(msg_hbm.at[wid, pl.ds(ebase, _SUP), :], msg_v)
            for c in range(_NCH):
                pltpu.sync_copy(msg_v.at[pl.ds(c * _CH, _CH), :],
                                acc_sh.at[didx_v.at[c]], add=True)
            return carry

        lax.fori_loop(0, nsup, super_body, 0)
        plsc.subcore_barrier()
        pltpu.sync_copy(acc_sh.at[pl.ds(rbase, nrows), :],
                        out_hbm.at[cid, pl.ds(rbase, nrows), :])

    return scatter_k


def kernel(x, edge_index, edge_features, W1, b1, W2, b2, W3, b3, W4, b4):
    bsz, n, d = x.shape
    e = edge_index.shape[1]
    de = edge_features.shape[1]
    assert bsz == 1 and d == 128 and de == 16
    assert e % (_NW * _SUP) == 0 and n % _NS == 0 and (n * de) % 128 == 0

    w1ab = jnp.concatenate([W1[:, :d], W1[:, d:2 * d]], axis=0)
    w1c_t = W1[:, 2 * d:].T
    eye8 = jnp.eye(8, dtype=jnp.float32)
    k1 = jnp.kron(eye8, w1c_t)
    b1t = jnp.tile(b1, 8)[None, :]
    k2 = jnp.kron(eye8, W2.T)
    b2t = jnp.tile(b2, 8)[None, :]
    w3a = W3[:, :d]
    w3b = W3[:, d:]
    b3t = b3[None, :]
    b4t = b4[None, :]

    x2d = x[0]
    ew = e // _NW
    sidx = edge_index[0].reshape(_NW, ew // _CH, _CH)
    didx = edge_index[1].reshape(_NW, ew // _CH, _CH)
    ef_p = edge_features.reshape(e // 8, 8 * de)
    n_pad = ((n + _NS * 8 - 1) // (_NS * 8)) * _NS * 8

    f32 = jnp.float32
    row_blk = 1000
    ep_blk = 2000
    n_grid = n // row_blk
    ep_grid = (e // 8) // ep_blk

    psrc, pdst = pl.pallas_call(
        _proj_nodes_body,
        grid=(n_grid,),
        in_specs=[
            pl.BlockSpec((row_blk, d), lambda i: (i, 0)),
            pl.BlockSpec((2 * de, d), lambda i: (0, 0)),
        ],
        out_specs=[
            pl.BlockSpec((row_blk, de), lambda i: (i, 0)),
            pl.BlockSpec((row_blk, de), lambda i: (i, 0)),
        ],
        out_shape=[jax.ShapeDtypeStruct((n, de), f32)] * 2,
    )(x2d, w1ab)

    eproj_p = pl.pallas_call(
        _edge_proj_body,
        grid=(ep_grid,),
        in_specs=[
            pl.BlockSpec((ep_blk, 8 * de), lambda i: (i, 0)),
            pl.BlockSpec((8 * de, 8 * de), lambda i: (0, 0)),
            pl.BlockSpec((1, 8 * de), lambda i: (0, 0)),
        ],
        out_specs=pl.BlockSpec((ep_blk, 8 * de), lambda i: (i, 0)),
        out_shape=jax.ShapeDtypeStruct((e // 8, 8 * de), f32),
    )(ef_p, k1, b1t)
    eproj = eproj_p.reshape(_NW, ew, de)

    h = _make_gather_kernel(n, e, de)(psrc, pdst, eproj, sidx, didx)

    msg_p = pl.pallas_call(
        _edge_mlp2_body,
        grid=(ep_grid,),
        in_specs=[
            pl.BlockSpec((ep_blk, 8 * de), lambda i: (i, 0)),
            pl.BlockSpec((8 * de, 8 * de), lambda i: (0, 0)),
            pl.BlockSpec((1, 8 * de), lambda i: (0, 0)),
        ],
        out_specs=pl.BlockSpec((ep_blk, 8 * de), lambda i: (i, 0)),
        out_shape=jax.ShapeDtypeStruct((e // 8, 8 * de), f32),
    )(h.reshape(e // 8, 8 * de), k2, b2t)
    msg = msg_p.reshape(_NW, ew, de)

    partials = _make_scatter_kernel(n_pad, e, de)(msg, didx)

    out2d = pl.pallas_call(
        _node_mlp_body,
        grid=(n_grid,),
        in_specs=[
            pl.BlockSpec((row_blk, d), lambda i: (i, 0)),
            pl.BlockSpec((_NC, row_blk, de), lambda i: (0, i, 0)),
            pl.BlockSpec((d, d), lambda i: (0, 0)),
            pl.BlockSpec((d, de), lambda i: (0, 0)),
            pl.BlockSpec((1, d), lambda i: (0, 0)),
            pl.BlockSpec((d, d), lambda i: (0, 0)),
            pl.BlockSpec((1, d), lambda i: (0, 0)),
        ],
        out_specs=pl.BlockSpec((row_blk, d), lambda i: (i, 0)),
        out_shape=jax.ShapeDtypeStruct((n, d), f32),
    )(x2d, partials, w3a, w3b, b3t, W4, b4t)

    return out2d[None]


# speedup vs baseline: 11.8397x; 1.0149x over previous
import functools

import jax
import jax.numpy as jnp
from jax import lax
from jax.experimental import pallas as pl
from jax.experimental.pallas import tpu as pltpu
from jax.experimental.pallas import tpu_sc as plsc

_NC = 2
_NS = 16
_NW = _NC * _NS

_SQRT_HALF = 0.7071067811865476


def _gelu(t):
    return 0.5 * t * (1.0 + lax.erf(t * _SQRT_HALF))


def _proj_nodes_body(x_ref, w_ref, psrc_ref, pdst_ref):
    p = lax.dot_general(x_ref[...], w_ref[...], (((1,), (1,)), ((), ())),
                        preferred_element_type=jnp.float32)
    de = psrc_ref.shape[-1]
    psrc_ref[...] = p[:, :de]
    pdst_ref[...] = p[:, de:]


def _edge_proj_body(ef_ref, k_ref, b_ref, out_ref):
    out_ref[...] = (
        lax.dot_general(ef_ref[...], k_ref[...], (((1,), (0,)), ((), ())),
                        preferred_element_type=jnp.float32)
        + b_ref[...]
    )


def _edge_mlp2_body(h_ref, k_ref, b_ref, out_ref):
    g = _gelu(h_ref[...])
    out_ref[...] = (
        lax.dot_general(g, k_ref[...], (((1,), (0,)), ((), ())),
                        preferred_element_type=jnp.float32)
        + b_ref[...]
    )


def _node_mlp_body(x_ref, p_ref, w3a_ref, w3b_ref, b3_ref, w4_ref, b4_ref,
                   out_ref):
    x = x_ref[...]
    agg = p_ref[0] + p_ref[1]
    t = (
        lax.dot_general(x, w3a_ref[...], (((1,), (1,)), ((), ())),
                        preferred_element_type=jnp.float32)
        + lax.dot_general(agg, w3b_ref[...], (((1,), (1,)), ((), ())),
                          preferred_element_type=jnp.float32)
        + b3_ref[...]
    )
    h2 = _gelu(t)
    out_ref[...] = (
        x
        + lax.dot_general(h2, w4_ref[...], (((1,), (1,)), ((), ())),
                          preferred_element_type=jnp.float32)
        + b4_ref[...]
    )


_SUP_E = 1024
_SUP_I = _SUP_E // 128
_SUP_P = _SUP_E // 8


def _make_gather_kernel(n_nodes, n_edges, de):
    rows_p = n_edges // 8
    n_sup = n_edges // _SUP_E
    tail_i = (n_edges - n_sup * _SUP_E) // 128
    base_cnt = n_sup // _NW
    extra = n_sup % _NW
    mesh = plsc.VectorSubcoreMesh(core_axis_name="c", subcore_axis_name="s",
                                  num_cores=_NC, num_subcores=_NS)

    @functools.partial(
        pl.kernel,
        mesh=mesh,
        out_type=jax.ShapeDtypeStruct((n_edges, de), jnp.float32),
        scratch_types=[
            pltpu.VMEM((_SUP_I, 128), jnp.int32),
            pltpu.VMEM((_SUP_I, 128), jnp.int32),
            pltpu.VMEM((_SUP_E, de), jnp.float32),
            pltpu.SemaphoreType.DMA,
        ],
        compiler_params=pltpu.CompilerParams(use_tc_tiling_on_sc=False),
    )
    def gather_k(psrc_hbm, pdst_hbm, eproj_hbm, sidx_hbm, didx_hbm, out_hbm,
                 sidx_v, didx_v, h_v, sem):
        wid = lax.axis_index("s") * _NC + lax.axis_index("c")
        cnt = jnp.where(wid < extra, base_cnt + 1, base_cnt)

        def run_super(ir, eb, n_i):
            pltpu.sync_copy(sidx_hbm.at[pl.ds(ir, n_i), :],
                            sidx_v.at[pl.ds(0, n_i), :])
            pltpu.sync_copy(didx_hbm.at[pl.ds(ir, n_i), :],
                            didx_v.at[pl.ds(0, n_i), :])
            pltpu.sync_copy(eproj_hbm.at[pl.ds(eb, n_i * 128), :],
                            h_v.at[pl.ds(0, n_i * 128), :])
            descs = []
            for c in range(n_i):
                descs.append(pltpu.async_copy(
                    psrc_hbm.at[sidx_v.at[c]],
                    h_v.at[pl.ds(c * 128, 128), :], sem, add=True))
            for d in descs:
                d.wait()
            descs = []
            for c in range(n_i):
                descs.append(pltpu.async_copy(
                    pdst_hbm.at[didx_v.at[c]],
                    h_v.at[pl.ds(c * 128, 128), :], sem, add=True))
            for d in descs:
                d.wait()
            pltpu.sync_copy(h_v.at[pl.ds(0, n_i * 128), :],
                            out_hbm.at[pl.ds(eb, n_i * 128), :])

        def super_body(k, carry):
            s = wid + k * _NW
            run_super(pl.multiple_of(s * _SUP_I, 8),
                      pl.multiple_of(s * _SUP_E, 8), _SUP_I)
            return carry

        lax.fori_loop(0, cnt, super_body, 0)
        if tail_i:
            @pl.when(wid == _NW - 1)
            def _():
                run_super(n_sup * _SUP_I, n_sup * _SUP_E, tail_i)

    return gather_k


def _make_scatter_kernel(n_pad, n_edges, de):
    n_sup = n_edges // _SUP_E
    tail_i = (n_edges - n_sup * _SUP_E) // 128
    base_cnt = n_sup // _NW
    extra = n_sup % _NW
    nrows = n_pad // _NS
    mesh = plsc.VectorSubcoreMesh(core_axis_name="c", subcore_axis_name="s",
                                  num_cores=_NC, num_subcores=_NS)

    @functools.partial(
        pl.kernel,
        mesh=mesh,
        out_type=jax.ShapeDtypeStruct((_NC, n_pad, de), jnp.float32),
        scratch_types=[
            pltpu.VMEM((_SUP_I, 128), jnp.int32),
            pltpu.VMEM((_SUP_E, de), jnp.float32),
            pltpu.VMEM_SHARED((n_pad, de), jnp.float32),
        ],
        compiler_params=pltpu.CompilerParams(use_tc_tiling_on_sc=False),
    )
    def scatter_k(msg_hbm, didx_hbm, out_hbm, didx_v, msg_v, acc_sh):
        cid = lax.axis_index("c")
        sid = lax.axis_index("s")
        wid = sid * _NC + cid
        cnt = jnp.where(wid < extra, base_cnt + 1, base_cnt)
        rbase = pl.multiple_of(sid * nrows, 8)

        def zrow(j, c):
            msg_v[j, :] = jnp.zeros((de,), jnp.float32)
            return c

        lax.fori_loop(0, nrows, zrow, 0)
        pltpu.sync_copy(msg_v.at[pl.ds(0, nrows), :],
                        acc_sh.at[pl.ds(rbase, nrows), :])
        plsc.subcore_barrier()

        def run_super(ir, eb, n_i):
            pltpu.sync_copy(didx_hbm.at[pl.ds(ir, n_i), :],
                            didx_v.at[pl.ds(0, n_i), :])
            pltpu.sync_copy(msg_hbm.at[pl.ds(eb, n_i * 128), :],
                            msg_v.at[pl.ds(0, n_i * 128), :])
            for c in range(n_i):
                pltpu.sync_copy(msg_v.at[pl.ds(c * 128, 128), :],
                                acc_sh.at[didx_v.at[c]], add=True)

        def super_body(k, carry):
            s = wid + k * _NW
            run_super(pl.multiple_of(s * _SUP_I, 8),
                      pl.multiple_of(s * _SUP_E, 8), _SUP_I)
            return carry

        lax.fori_loop(0, cnt, super_body, 0)
        if tail_i:
            @pl.when(wid == _NW - 1)
            def _():
                run_super(n_sup * _SUP_I, n_sup * _SUP_E, tail_i)
        plsc.subcore_barrier()
        pltpu.sync_copy(acc_sh.at[pl.ds(rbase, nrows), :],
                        out_hbm.at[cid, pl.ds(rbase, nrows), :])

    return scatter_k


def kernel(x, edge_index, edge_features, W1, b1, W2, b2, W3, b3, W4, b4):
    bsz, n, d = x.shape
    e = edge_index.shape[1]
    de = edge_features.shape[1]
    assert bsz == 1 and d == 128 and de == 16
    assert e % 1024 in (0, 512) and e % 128 == 0 and n % _NS == 0

    w1ab = jnp.concatenate([W1[:, :d], W1[:, d:2 * d]], axis=0)
    w1c_t = W1[:, 2 * d:].T
    eye8 = jnp.eye(8, dtype=jnp.float32)
    k1 = jnp.kron(eye8, w1c_t)
    b1t = jnp.tile(b1, 8)[None, :]
    k2 = jnp.kron(eye8, W2.T)
    b2t = jnp.tile(b2, 8)[None, :]
    w3a = W3[:, :d]
    w3b = W3[:, d:]
    b3t = b3[None, :]
    b4t = b4[None, :]

    x2d = x[0]
    sidx = edge_index[0].reshape(e // 128, 128)
    didx = edge_index[1].reshape(e // 128, 128)
    ef_p = edge_features.reshape(e // 8, 8 * de)
    n_pad = ((n + _NS * 8 - 1) // (_NS * 8)) * _NS * 8

    f32 = jnp.float32
    row_blk = 1000
    ep_blk = 2000
    n_grid = n // row_blk
    ep_grid = (e // 8) // ep_blk

    psrc, pdst = pl.pallas_call(
        _proj_nodes_body,
        grid=(n_grid,),
        in_specs=[
            pl.BlockSpec((row_blk, d), lambda i: (i, 0)),
            pl.BlockSpec((2 * de, d), lambda i: (0, 0)),
        ],
        out_specs=[
            pl.BlockSpec((row_blk, de), lambda i: (i, 0)),
            pl.BlockSpec((row_blk, de), lambda i: (i, 0)),
        ],
        out_shape=[jax.ShapeDtypeStruct((n, de), f32)] * 2,
    )(x2d, w1ab)

    eproj_p = pl.pallas_call(
        _edge_proj_body,
        grid=(ep_grid,),
        in_specs=[
            pl.BlockSpec((ep_blk, 8 * de), lambda i: (i, 0)),
            pl.BlockSpec((8 * de, 8 * de), lambda i: (0, 0)),
            pl.BlockSpec((1, 8 * de), lambda i: (0, 0)),
        ],
        out_specs=pl.BlockSpec((ep_blk, 8 * de), lambda i: (i, 0)),
        out_shape=jax.ShapeDtypeStruct((e // 8, 8 * de), f32),
    )(ef_p, k1, b1t)

    h = _make_gather_kernel(n, e, de)(psrc, pdst, eproj_p.reshape(e, de),
                                      sidx, didx)
    h_p = h.reshape(e // 8, 8 * de)

    msg_p = pl.pallas_call(
        _edge_mlp2_body,
        grid=(ep_grid,),
        in_specs=[
            pl.BlockSpec((ep_blk, 8 * de), lambda i: (i, 0)),
            pl.BlockSpec((8 * de, 8 * de), lambda i: (0, 0)),
            pl.BlockSpec((1, 8 * de), lambda i: (0, 0)),
        ],
        out_specs=pl.BlockSpec((ep_blk, 8 * de), lambda i: (i, 0)),
        out_shape=jax.ShapeDtypeStruct((e // 8, 8 * de), f32),
    )(h_p, k2, b2t)

    partials = _make_scatter_kernel(n_pad, e, de)(msg_p.reshape(e, de), didx)

    out2d = pl.pallas_call(
        _node_mlp_body,
        grid=(n_grid,),
        in_specs=[
            pl.BlockSpec((row_blk, d), lambda i: (i, 0)),
            pl.BlockSpec((_NC, row_blk, de), lambda i: (0, i, 0)),
            pl.BlockSpec((d, d), lambda i: (0, 0)),
            pl.BlockSpec((d, de), lambda i: (0, 0)),
            pl.BlockSpec((1, d), lambda i: (0, 0)),
            pl.BlockSpec((d, d), lambda i: (0, 0)),
            pl.BlockSpec((1, d), lambda i: (0, 0)),
        ],
        out_specs=pl.BlockSpec((row_blk, d), lambda i: (i, 0)),
        out_shape=jax.ShapeDtypeStruct((n, d), f32),
    )(x2d, partials, w3a, w3b, b3t, W4, b4t)

    return out2d[None]


# speedup vs baseline: 15.0722x; 1.2730x over previous
import functools

import jax
import jax.numpy as jnp
from jax import lax
from jax.experimental import pallas as pl
from jax.experimental.pallas import tpu as pltpu
from jax.experimental.pallas import tpu_sc as plsc

_NC = 2
_NS = 16
_NW = _NC * _NS

_SQRT_HALF = 0.7071067811865476


def _gelu(t):
    return 0.5 * t * (1.0 + lax.erf(t * _SQRT_HALF))


def _proj_nodes_body(x_ref, w_ref, psrc_ref, pdst_ref):
    p = lax.dot_general(x_ref[...], w_ref[...], (((1,), (1,)), ((), ())),
                        preferred_element_type=jnp.float32)
    de = psrc_ref.shape[-1]
    psrc_ref[...] = p[:, :de]
    pdst_ref[...] = p[:, de:]


def _edge_proj_body(ef_ref, k_ref, b_ref, out_ref):
    out_ref[...] = (
        lax.dot_general(ef_ref[...], k_ref[...], (((1,), (0,)), ((), ())),
                        preferred_element_type=jnp.float32)
        + b_ref[...]
    )


def _edge_mlp2_body(h_ref, ef_ref, k1_ref, b1_ref, k2_ref, b2_ref, out_ref):
    t = (
        h_ref[...]
        + lax.dot_general(ef_ref[...], k1_ref[...], (((1,), (0,)), ((), ())),
                          preferred_element_type=jnp.float32)
        + b1_ref[...]
    )
    out_ref[...] = (
        lax.dot_general(_gelu(t), k2_ref[...], (((1,), (0,)), ((), ())),
                        preferred_element_type=jnp.float32)
        + b2_ref[...]
    )


def _node_mlp_body(x_ref, p_ref, w3a_ref, w3b_ref, b3_ref, w4_ref, b4_ref,
                   out_ref):
    x = x_ref[...]
    agg = p_ref[0] + p_ref[1]
    t = (
        lax.dot_general(x, w3a_ref[...], (((1,), (1,)), ((), ())),
                        preferred_element_type=jnp.float32)
        + lax.dot_general(agg, w3b_ref[...], (((1,), (1,)), ((), ())),
                          preferred_element_type=jnp.float32)
        + b3_ref[...]
    )
    h2 = _gelu(t)
    out_ref[...] = (
        x
        + lax.dot_general(h2, w4_ref[...], (((1,), (1,)), ((), ())),
                          preferred_element_type=jnp.float32)
        + b4_ref[...]
    )


_SUP_E = 1024
_SUP_I = _SUP_E // 128
_SUP_P = _SUP_E // 8


def _make_gather_kernel(n_nodes, n_edges, de):
    rows_p = n_edges // 8
    n_sup = n_edges // _SUP_E
    tail_i = (n_edges - n_sup * _SUP_E) // 128
    base_cnt = n_sup // _NW
    extra = n_sup % _NW
    mesh = plsc.VectorSubcoreMesh(core_axis_name="c", subcore_axis_name="s",
                                  num_cores=_NC, num_subcores=_NS)

    @functools.partial(
        pl.kernel,
        mesh=mesh,
        out_type=jax.ShapeDtypeStruct((n_edges, de), jnp.float32),
        scratch_types=[
            pltpu.VMEM((_SUP_E,), jnp.int32),
            pltpu.VMEM((_SUP_E,), jnp.int32),
            pltpu.VMEM((_SUP_E, de), jnp.float32),
            pltpu.SemaphoreType.DMA,
        ],
        compiler_params=pltpu.CompilerParams(use_tc_tiling_on_sc=False),
    )
    def gather_k(psrc_hbm, pdst_hbm, sidx_hbm, didx_hbm, out_hbm,
                 sidx_v, didx_v, h_v, sem):
        wid = lax.axis_index("s") * _NC + lax.axis_index("c")
        cnt = jnp.where(wid < extra, base_cnt + 1, base_cnt)

        def run_super(eb, n_i):
            pltpu.sync_copy(sidx_hbm.at[pl.ds(eb, n_i * 128)],
                            sidx_v.at[pl.ds(0, n_i * 128)])
            pltpu.sync_copy(didx_hbm.at[pl.ds(eb, n_i * 128)],
                            didx_v.at[pl.ds(0, n_i * 128)])
            descs = []
            for c in range(n_i):
                descs.append(pltpu.async_copy(
                    psrc_hbm.at[sidx_v.at[pl.ds(c * 128, 128)]],
                    h_v.at[pl.ds(c * 128, 128), :], sem))
            for d in descs:
                d.wait()
            descs = []
            for c in range(n_i):
                descs.append(pltpu.async_copy(
                    pdst_hbm.at[didx_v.at[pl.ds(c * 128, 128)]],
                    h_v.at[pl.ds(c * 128, 128), :], sem, add=True))
            for d in descs:
                d.wait()
            pltpu.sync_copy(h_v.at[pl.ds(0, n_i * 128), :],
                            out_hbm.at[pl.ds(eb, n_i * 128), :])

        def super_body(k, carry):
            s = wid + k * _NW
            run_super(pl.multiple_of(s * _SUP_E, 8), _SUP_I)
            return carry

        lax.fori_loop(0, cnt, super_body, 0)
        if tail_i:
            @pl.when(wid == _NW - 1)
            def _():
                run_super(n_sup * _SUP_E, tail_i)

    return gather_k


def _make_scatter_kernel(n_pad, n_edges, de):
    n_sup = n_edges // _SUP_E
    tail_i = (n_edges - n_sup * _SUP_E) // 128
    base_cnt = n_sup // _NW
    extra = n_sup % _NW
    nrows = n_pad // _NS
    mesh = plsc.VectorSubcoreMesh(core_axis_name="c", subcore_axis_name="s",
                                  num_cores=_NC, num_subcores=_NS)

    @functools.partial(
        pl.kernel,
        mesh=mesh,
        out_type=jax.ShapeDtypeStruct((_NC, n_pad, de), jnp.float32),
        scratch_types=[
            pltpu.VMEM((_SUP_I, 128), jnp.int32),
            pltpu.VMEM((_SUP_E, de), jnp.float32),
            pltpu.VMEM_SHARED((n_pad, de), jnp.float32),
        ],
        compiler_params=pltpu.CompilerParams(use_tc_tiling_on_sc=False),
    )
    def scatter_k(msg_hbm, didx_hbm, out_hbm, didx_v, msg_v, acc_sh):
        cid = lax.axis_index("c")
        sid = lax.axis_index("s")
        wid = sid * _NC + cid
        cnt = jnp.where(wid < extra, base_cnt + 1, base_cnt)
        rbase = pl.multiple_of(sid * nrows, 8)

        def zrow(j, c):
            msg_v[j, :] = jnp.zeros((de,), jnp.float32)
            return c

        lax.fori_loop(0, nrows, zrow, 0)
        pltpu.sync_copy(msg_v.at[pl.ds(0, nrows), :],
                        acc_sh.at[pl.ds(rbase, nrows), :])
        plsc.subcore_barrier()

        def run_super(ir, eb, n_i):
            pltpu.sync_copy(didx_hbm.at[pl.ds(ir, n_i), :],
                            didx_v.at[pl.ds(0, n_i), :])
            pltpu.sync_copy(msg_hbm.at[pl.ds(eb, n_i * 128), :],
                            msg_v.at[pl.ds(0, n_i * 128), :])
            for c in range(n_i):
                pltpu.sync_copy(msg_v.at[pl.ds(c * 128, 128), :],
                                acc_sh.at[didx_v.at[c]], add=True)

        def super_body(k, carry):
            s = wid + k * _NW
            run_super(pl.multiple_of(s * _SUP_I, 8),
                      pl.multiple_of(s * _SUP_E, 8), _SUP_I)
            return carry

        lax.fori_loop(0, cnt, super_body, 0)
        if tail_i:
            @pl.when(wid == _NW - 1)
            def _():
                run_super(n_sup * _SUP_I, n_sup * _SUP_E, tail_i)
        plsc.subcore_barrier()
        pltpu.sync_copy(acc_sh.at[pl.ds(rbase, nrows), :],
                        out_hbm.at[cid, pl.ds(rbase, nrows), :])

    return scatter_k


def kernel(x, edge_index, edge_features, W1, b1, W2, b2, W3, b3, W4, b4):
    bsz, n, d = x.shape
    e = edge_index.shape[1]
    de = edge_features.shape[1]
    assert bsz == 1 and d == 128 and de == 16
    assert e % 1024 in (0, 512) and e % 128 == 0 and n % _NS == 0

    w1ab = jnp.concatenate([W1[:, :d], W1[:, d:2 * d]], axis=0)
    w1c_t = W1[:, 2 * d:].T
    eye8 = jnp.eye(8, dtype=jnp.float32)
    k1 = jnp.kron(eye8, w1c_t)
    b1t = jnp.tile(b1, 8)[None, :]
    k2 = jnp.kron(eye8, W2.T)
    b2t = jnp.tile(b2, 8)[None, :]
    w3a = W3[:, :d]
    w3b = W3[:, d:]
    b3t = b3[None, :]
    b4t = b4[None, :]

    x2d = x[0]
    sidx1 = edge_index[0]
    didx1 = edge_index[1]
    didx = edge_index[1].reshape(e // 128, 128)
    ef_p = edge_features.reshape(e // 8, 8 * de)
    n_pad = ((n + _NS * 8 - 1) // (_NS * 8)) * _NS * 8

    f32 = jnp.float32
    row_blk = 1000
    ep_blk = 2000
    n_grid = n // row_blk
    ep_grid = (e // 8) // ep_blk

    psrc, pdst = pl.pallas_call(
        _proj_nodes_body,
        grid=(n_grid,),
        in_specs=[
            pl.BlockSpec((row_blk, d), lambda i: (i, 0)),
            pl.BlockSpec((2 * de, d), lambda i: (0, 0)),
        ],
        out_specs=[
            pl.BlockSpec((row_blk, de), lambda i: (i, 0)),
            pl.BlockSpec((row_blk, de), lambda i: (i, 0)),
        ],
        out_shape=[jax.ShapeDtypeStruct((n, de), f32)] * 2,
    )(x2d, w1ab)

    h = _make_gather_kernel(n, e, de)(psrc, pdst, sidx1, didx1)
    h_p = h.reshape(e // 8, 8 * de)

    msg_p = pl.pallas_call(
        _edge_mlp2_body,
        grid=(ep_grid,),
        in_specs=[
            pl.BlockSpec((ep_blk, 8 * de), lambda i: (i, 0)),
            pl.BlockSpec((ep_blk, 8 * de), lambda i: (i, 0)),
            pl.BlockSpec((8 * de, 8 * de), lambda i: (0, 0)),
            pl.BlockSpec((1, 8 * de), lambda i: (0, 0)),
            pl.BlockSpec((8 * de, 8 * de), lambda i: (0, 0)),
            pl.BlockSpec((1, 8 * de), lambda i: (0, 0)),
        ],
        out_specs=pl.BlockSpec((ep_blk, 8 * de), lambda i: (i, 0)),
        out_shape=jax.ShapeDtypeStruct((e // 8, 8 * de), f32),
    )(h_p, ef_p, k1, b1t, k2, b2t)

    partials = _make_scatter_kernel(n_pad, e, de)(msg_p.reshape(e, de), didx)

    out2d = pl.pallas_call(
        _node_mlp_body,
        grid=(n_grid,),
        in_specs=[
            pl.BlockSpec((row_blk, d), lambda i: (i, 0)),
            pl.BlockSpec((_NC, row_blk, de), lambda i: (0, i, 0)),
            pl.BlockSpec((d, d), lambda i: (0, 0)),
            pl.BlockSpec((d, de), lambda i: (0, 0)),
            pl.BlockSpec((1, d), lambda i: (0, 0)),
            pl.BlockSpec((d, d), lambda i: (0, 0)),
            pl.BlockSpec((1, d), lambda i: (0, 0)),
        ],
        out_specs=pl.BlockSpec((row_blk, d), lambda i: (i, 0)),
        out_shape=jax.ShapeDtypeStruct((n, d), f32),
    )(x2d, partials, w3a, w3b, b3t, W4, b4t)

    return out2d[None]


# speedup vs baseline: 15.4393x; 1.0244x over previous
import functools

import jax
import jax.numpy as jnp
from jax import lax
from jax.experimental import pallas as pl
from jax.experimental.pallas import tpu as pltpu
from jax.experimental.pallas import tpu_sc as plsc

_NC = 2
_NS = 16
_NW = _NC * _NS

_SQRT_HALF = 0.7071067811865476


def _gelu(t):
    return 0.5 * t * (1.0 + lax.erf(t * _SQRT_HALF))


def _proj_nodes_body(x_ref, w_ref, psrc_ref, pdst_ref):
    p = lax.dot_general(x_ref[...], w_ref[...], (((1,), (1,)), ((), ())),
                        preferred_element_type=jnp.float32)
    de = psrc_ref.shape[-1]
    psrc_ref[...] = p[:, :de]
    pdst_ref[...] = p[:, de:]


def _edge_proj_body(ef_ref, k_ref, b_ref, out_ref):
    out_ref[...] = (
        lax.dot_general(ef_ref[...], k_ref[...], (((1,), (0,)), ((), ())),
                        preferred_element_type=jnp.float32)
        + b_ref[...]
    )


def _edge_mlp2_body(h_ref, ef_ref, k1_ref, b1_ref, k2_ref, b2_ref, out_ref):
    t = (
        h_ref[...]
        + lax.dot_general(ef_ref[...], k1_ref[...], (((1,), (0,)), ((), ())),
                          preferred_element_type=jnp.float32)
        + b1_ref[...]
    )
    out_ref[...] = (
        lax.dot_general(_gelu(t), k2_ref[...], (((1,), (0,)), ((), ())),
                        preferred_element_type=jnp.float32)
        + b2_ref[...]
    )


def _node_mlp_body(x_ref, p_ref, w3a_ref, w3b_ref, b3_ref, w4_ref, b4_ref,
                   out_ref):
    x = x_ref[...]
    agg = p_ref[0] + p_ref[1]
    t = (
        lax.dot_general(x, w3a_ref[...], (((1,), (1,)), ((), ())),
                        preferred_element_type=jnp.float32)
        + lax.dot_general(agg, w3b_ref[...], (((1,), (1,)), ((), ())),
                          preferred_element_type=jnp.float32)
        + b3_ref[...]
    )
    h2 = _gelu(t)
    out_ref[...] = (
        x
        + lax.dot_general(h2, w4_ref[...], (((1,), (1,)), ((), ())),
                          preferred_element_type=jnp.float32)
        + b4_ref[...]
    )


_SUP_E = 1024
_SUP_I = _SUP_E // 128
_SUP_P = _SUP_E // 8


def _make_gather_kernel(n_nodes, n_edges, de):
    rows_p = n_edges // 8
    n_sup = n_edges // _SUP_E
    tail_i = (n_edges - n_sup * _SUP_E) // 128
    base_cnt = n_sup // _NW
    extra = n_sup % _NW
    mesh = plsc.VectorSubcoreMesh(core_axis_name="c", subcore_axis_name="s",
                                  num_cores=_NC, num_subcores=_NS)

    @functools.partial(
        pl.kernel,
        mesh=mesh,
        out_type=jax.ShapeDtypeStruct((n_edges, de), jnp.float32),
        scratch_types=[
            pltpu.VMEM((_SUP_E,), jnp.int32),
            pltpu.VMEM((_SUP_E,), jnp.int32),
            pltpu.VMEM((_SUP_E, de), jnp.float32),
            pltpu.SemaphoreType.DMA,
        ],
        compiler_params=pltpu.CompilerParams(use_tc_tiling_on_sc=False),
    )
    def gather_k(psrc_hbm, pdst_hbm, eidx_hbm, out_hbm,
                 sidx_v, didx_v, h_v, sem):
        wid = lax.axis_index("s") * _NC + lax.axis_index("c")
        cnt = jnp.where(wid < extra, base_cnt + 1, base_cnt)

        def run_super(eb, n_i):
            pltpu.sync_copy(eidx_hbm.at[0, pl.ds(eb, n_i * 128)],
                            sidx_v.at[pl.ds(0, n_i * 128)])
            pltpu.sync_copy(eidx_hbm.at[1, pl.ds(eb, n_i * 128)],
                            didx_v.at[pl.ds(0, n_i * 128)])
            descs = []
            for c in range(n_i):
                descs.append(pltpu.async_copy(
                    psrc_hbm.at[sidx_v.at[pl.ds(c * 128, 128)]],
                    h_v.at[pl.ds(c * 128, 128), :], sem))
            for d in descs:
                d.wait()
            descs = []
            for c in range(n_i):
                descs.append(pltpu.async_copy(
                    pdst_hbm.at[didx_v.at[pl.ds(c * 128, 128)]],
                    h_v.at[pl.ds(c * 128, 128), :], sem, add=True))
            for d in descs:
                d.wait()
            pltpu.sync_copy(h_v.at[pl.ds(0, n_i * 128), :],
                            out_hbm.at[pl.ds(eb, n_i * 128), :])

        def super_body(k, carry):
            s = wid + k * _NW
            run_super(pl.multiple_of(s * _SUP_E, 8), _SUP_I)
            return carry

        lax.fori_loop(0, cnt, super_body, 0)
        if tail_i:
            @pl.when(wid == _NW - 1)
            def _():
                run_super(n_sup * _SUP_E, tail_i)

    return gather_k


def _make_scatter_kernel(n_pad, n_edges, de):
    n_sup = n_edges // _SUP_E
    tail_i = (n_edges - n_sup * _SUP_E) // 128
    base_cnt = n_sup // _NW
    extra = n_sup % _NW
    nrows = n_pad // _NS
    mesh = plsc.VectorSubcoreMesh(core_axis_name="c", subcore_axis_name="s",
                                  num_cores=_NC, num_subcores=_NS)

    @functools.partial(
        pl.kernel,
        mesh=mesh,
        out_type=jax.ShapeDtypeStruct((_NC, n_pad, de), jnp.float32),
        scratch_types=[
            pltpu.VMEM((_SUP_I, 128), jnp.int32),
            pltpu.VMEM((_SUP_E, de), jnp.float32),
            pltpu.VMEM_SHARED((n_pad, de), jnp.float32),
            pltpu.SemaphoreType.DMA,
        ],
        compiler_params=pltpu.CompilerParams(use_tc_tiling_on_sc=False),
    )
    def scatter_k(msg_hbm, didx_hbm, out_hbm, didx_v, msg_v, acc_sh, sem):
        cid = lax.axis_index("c")
        sid = lax.axis_index("s")
        wid = sid * _NC + cid
        cnt = jnp.where(wid < extra, base_cnt + 1, base_cnt)
        rbase = pl.multiple_of(sid * nrows, 8)

        def zrow(j, c):
            msg_v[j, :] = jnp.zeros((de,), jnp.float32)
            return c

        lax.fori_loop(0, nrows, zrow, 0)
        pltpu.sync_copy(msg_v.at[pl.ds(0, nrows), :],
                        acc_sh.at[pl.ds(rbase, nrows), :])
        plsc.subcore_barrier()

        def run_super(ir, eb, n_i):
            pltpu.sync_copy(didx_hbm.at[pl.ds(ir, n_i), :],
                            didx_v.at[pl.ds(0, n_i), :])
            pltpu.sync_copy(msg_hbm.at[pl.ds(eb, n_i * 128), :],
                            msg_v.at[pl.ds(0, n_i * 128), :])
            descs = []
            for c in range(n_i):
                descs.append(pltpu.async_copy(
                    msg_v.at[pl.ds(c * 128, 128), :],
                    acc_sh.at[didx_v.at[c]], sem, add=True))
            for d in descs:
                d.wait()

        def super_body(k, carry):
            s = wid + k * _NW
            run_super(pl.multiple_of(s * _SUP_I, 8),
                      pl.multiple_of(s * _SUP_E, 8), _SUP_I)
            return carry

        lax.fori_loop(0, cnt, super_body, 0)
        if tail_i:
            @pl.when(wid == _NW - 1)
            def _():
                run_super(n_sup * _SUP_I, n_sup * _SUP_E, tail_i)
        plsc.subcore_barrier()
        pltpu.sync_copy(acc_sh.at[pl.ds(rbase, nrows), :],
                        out_hbm.at[cid, pl.ds(rbase, nrows), :])

    return scatter_k


def kernel(x, edge_index, edge_features, W1, b1, W2, b2, W3, b3, W4, b4):
    bsz, n, d = x.shape
    e = edge_index.shape[1]
    de = edge_features.shape[1]
    assert bsz == 1 and d == 128 and de == 16
    assert e % 1024 in (0, 512) and e % 128 == 0 and n % _NS == 0

    w1ab = jnp.concatenate([W1[:, :d], W1[:, d:2 * d]], axis=0)
    w1c_t = W1[:, 2 * d:].T
    eye8 = jnp.eye(8, dtype=jnp.float32)
    k1 = jnp.kron(eye8, w1c_t)
    b1t = jnp.tile(b1, 8)[None, :]
    k2 = jnp.kron(eye8, W2.T)
    b2t = jnp.tile(b2, 8)[None, :]
    w3a = W3[:, :d]
    w3b = W3[:, d:]
    b3t = b3[None, :]
    b4t = b4[None, :]

    x2d = x[0]
    didx = edge_index[1].reshape(e // 128, 128)
    ef_p = edge_features.reshape(e // 8, 8 * de)
    n_pad = ((n + _NS * 8 - 1) // (_NS * 8)) * _NS * 8

    f32 = jnp.float32
    row_blk = 1000
    a1_blk = 2000
    ep_blk = 2000
    n_grid = n // row_blk
    a1_grid = n // a1_blk
    ep_grid = (e // 8) // ep_blk

    psrc, pdst = pl.pallas_call(
        _proj_nodes_body,
        grid=(a1_grid,),
        in_specs=[
            pl.BlockSpec((a1_blk, d), lambda i: (i, 0)),
            pl.BlockSpec((2 * de, d), lambda i: (0, 0)),
        ],
        out_specs=[
            pl.BlockSpec((a1_blk, de), lambda i: (i, 0)),
            pl.BlockSpec((a1_blk, de), lambda i: (i, 0)),
        ],
        out_shape=[jax.ShapeDtypeStruct((n, de), f32)] * 2,
    )(x2d, w1ab)

    h = _make_gather_kernel(n, e, de)(psrc, pdst, edge_index)
    h_p = h.reshape(e // 8, 8 * de)

    msg_p = pl.pallas_call(
        _edge_mlp2_body,
        grid=(ep_grid,),
        in_specs=[
            pl.BlockSpec((ep_blk, 8 * de), lambda i: (i, 0)),
            pl.BlockSpec((ep_blk, 8 * de), lambda i: (i, 0)),
            pl.BlockSpec((8 * de, 8 * de), lambda i: (0, 0)),
            pl.BlockSpec((1, 8 * de), lambda i: (0, 0)),
            pl.BlockSpec((8 * de, 8 * de), lambda i: (0, 0)),
            pl.BlockSpec((1, 8 * de), lambda i: (0, 0)),
        ],
        out_specs=pl.BlockSpec((ep_blk, 8 * de), lambda i: (i, 0)),
        out_shape=jax.ShapeDtypeStruct((e // 8, 8 * de), f32),
    )(h_p, ef_p, k1, b1t, k2, b2t)

    partials = _make_scatter_kernel(n_pad, e, de)(msg_p.reshape(e, de), didx)

    out2d = pl.pallas_call(
        _node_mlp_body,
        grid=(n_grid,),
        in_specs=[
            pl.BlockSpec((row_blk, d), lambda i: (i, 0)),
            pl.BlockSpec((_NC, row_blk, de), lambda i: (0, i, 0)),
            pl.BlockSpec((d, d), lambda i: (0, 0)),
            pl.BlockSpec((d, de), lambda i: (0, 0)),
            pl.BlockSpec((1, d), lambda i: (0, 0)),
            pl.BlockSpec((d, d), lambda i: (0, 0)),
            pl.BlockSpec((1, d), lambda i: (0, 0)),
        ],
        out_specs=pl.BlockSpec((row_blk, d), lambda i: (i, 0)),
        out_shape=jax.ShapeDtypeStruct((n, d), f32),
    )(x2d, partials, w3a, w3b, b3t, W4, b4t)

    return out2d[None]
